# Initial kernel scaffold; baseline (speedup 1.0000x reference)
import jax
import jax.numpy as jnp
from jax.experimental import pallas as pl


def kernel(states, edge_ids, W, Bt, K, RK, Bg):
    raise NotImplementedError("write your pallas kernel here")


# speedup vs baseline: 2.7043x; 2.7043x over previous
import functools

import jax
import jax.numpy as jnp
from jax import lax
from jax.experimental import pallas as pl
from jax.experimental.pallas import tpu as pltpu
from jax.experimental.pallas import tpu_sc as plsc

N_NODES = 10000
HIDDEN = 256
HX = 272
BN = 1000
E = 160000
NW = 32
EW = E // NW
EWP = EW + 16
NT = 8
QN = N_NODES // 4
CHUNK = 64
NCH = 14
SLABR = 2560
JUNK = 2500
ZR = 128

_mesh = plsc.VectorSubcoreMesh(core_axis_name="c", subcore_axis_name="s")
_i32 = jnp.int32
_f32 = jnp.float32


def _prep_body(et_hbm, sr_hbm, tg_hbm, srcl_out, locl_out, nch_out,
               et_v, sr_v, tg_v, sbuf, lbuf, nbuf):
    w = lax.axis_index("s") * 2 + lax.axis_index("c")
    base = w * EW
    pltpu.sync_copy(et_hbm.at[pl.ds(base, EWP)], et_v)
    pltpu.sync_copy(sr_hbm.at[pl.ds(base, EWP)], sr_v)
    pltpu.sync_copy(tg_hbm.at[pl.ds(base, EWP)], tg_v)

    lane = lax.iota(_i32, 16)
    nsteps = EWP // 16

    def make_step(tbase):
        def step(i, cnts):
            off = i * 16
            et = et_v[pl.ds(off, 16)]
            sr = sr_v[pl.ds(off, 16)]
            tg = tg_v[pl.ds(off, 16)]
            valid = (off + lane) < EW
            q = tg // QN
            tsk = et * 4 + q
            loc = tg - q * QN
            new = []
            for tt in range(2):
                t = tbase + tt
                m = valid & (tsk == t)
                mi = jnp.where(m, 1, 0)
                cs = lax.cumsum(mi)
                pos = cnts[tt] + cs - mi
                pk = pos // 16
                pcol = pos % 16
                t16 = jnp.full((16,), t, _i32)
                plsc.store_scatter(sbuf, [t16, pk, pcol], sr, mask=m)
                plsc.store_scatter(lbuf, [t16, pk, pcol], loc, mask=m)
                new.append(cnts[tt] + jnp.sum(mi))
            return tuple(new)
        return step

    z = jnp.asarray(0, _i32)
    cnts = (lax.fori_loop(0, nsteps, make_step(0), (z, z))
            + lax.fori_loop(0, nsteps, make_step(2), (z, z))
            + lax.fori_loop(0, nsteps, make_step(4), (z, z))
            + lax.fori_loop(0, nsteps, make_step(6), (z, z)))

    nch_vec = jnp.zeros((16,), _i32)
    zeros16 = jnp.zeros((16,), _i32)
    junk16 = jnp.full((16,), JUNK, _i32)
    for t in range(NT):
        cnt = cnts[t]
        nch = (cnt + (CHUNK - 1)) // CHUNK
        end = nch * CHUNK
        t16 = jnp.full((16,), t, _i32)

        def padp(p, carry, cnt=cnt, end=end, t16=t16):
            idx = cnt + p * 16 + lane
            m = idx < end
            pk = idx // 16
            pcol = idx % 16
            plsc.store_scatter(sbuf, [t16, pk, pcol], zeros16, mask=m)
            plsc.store_scatter(lbuf, [t16, pk, pcol], junk16, mask=m)
            return carry

        lax.fori_loop(0, CHUNK // 16, padp, jnp.asarray(0, _i32))
        nch_vec = jnp.where(lane == t, nch, nch_vec)

    nbuf[...] = nch_vec
    pltpu.sync_copy(nbuf, nch_out.at[w])
    pltpu.sync_copy(sbuf, srcl_out.at[w])
    pltpu.sync_copy(lbuf, locl_out.at[w])


_prep = functools.partial(
    pl.kernel,
    out_type=[
        jax.ShapeDtypeStruct((NW, NT, NCH * 4, 16), _i32),
        jax.ShapeDtypeStruct((NW, NT, NCH * 4, 16), _i32),
        jax.ShapeDtypeStruct((NW, 16), _i32),
    ],
    mesh=_mesh,
    compiler_params=pltpu.CompilerParams(needs_layout_passes=False),
    scratch_types=[
        pltpu.VMEM((EWP,), _i32),
        pltpu.VMEM((EWP,), _i32),
        pltpu.VMEM((EWP,), _i32),
        pltpu.VMEM((NT, NCH * 4, 16), _i32),
        pltpu.VMEM((NT, NCH * 4, 16), _i32),
        pltpu.VMEM((16,), _i32),
    ],
)(_prep_body)


def _accum_body(hx_hbm, srcl_hbm, locl_hbm, nch_hbm, z_hbm, acc_out,
                table, slab, g0, g1, sidx, lidx, cbuf, sem0, sem1):
    c = lax.axis_index("c")
    s = lax.axis_index("s")
    lane = lax.iota(_i32, 16)
    gbufs = (g0, g1)
    sems = (sem0, sem1)

    for (c0, cw) in ((0, 136), (136, 136)):
        tr = s * 624
        pltpu.sync_copy(hx_hbm.at[pl.ds(tr, 624), pl.ds(c0, cw)],
                        table.at[pl.ds(tr, 624), pl.ds(0, cw)])
        pl.when(s == 0)(lambda c0=c0, cw=cw: pltpu.sync_copy(
            hx_hbm.at[pl.ds(9984, 16), pl.ds(c0, cw)],
            table.at[pl.ds(9984, 16), pl.ds(0, cw)]))
        plsc.subcore_barrier()

        for q in range(4):
            r0 = s * 160
            pltpu.sync_copy(z_hbm.at[pl.ds(0, 128)], slab.at[pl.ds(r0, 128)])
            pltpu.sync_copy(z_hbm.at[pl.ds(0, 32)],
                            slab.at[pl.ds(r0 + 128, 32)])
            plsc.subcore_barrier()

            j = c * 4 + q
            for wsel in range(2):
                w = s * 2 + wsel
                pltpu.sync_copy(nch_hbm.at[w], cbuf)
                n = jnp.sum(jnp.where(lane == j, cbuf[...], 0))
                pltpu.sync_copy(srcl_hbm.at[w, j], sidx)
                pltpu.sync_copy(locl_hbm.at[w, j], lidx)

                descs = [
                    pltpu.make_async_copy(table.at[sidx.at[k]],
                                          gbufs[k % 2], sems[k % 2])
                    for k in range(NCH)
                ]
                pl.when(n > 0)(lambda: descs[0].start())
                for k in range(NCH):
                    if k + 1 < NCH:
                        pl.when(k + 1 < n)(lambda k=k: descs[k + 1].start())

                    def chunk(k=k):
                        descs[k].wait()
                        pltpu.sync_copy(gbufs[k % 2], slab.at[lidx.at[k]],
                                        add=True)
                    pl.when(k < n)(chunk)

            plsc.subcore_barrier()
            wr0 = s * 152
            pltpu.sync_copy(
                slab.at[pl.ds(wr0, 152), pl.ds(0, cw)],
                acc_out.at[c].at[pl.ds(q * QN + wr0, 152), pl.ds(c0, cw)])
            pl.when(s == 0)(lambda c0=c0, cw=cw, q=q: pltpu.sync_copy(
                slab.at[pl.ds(2432, 68), pl.ds(0, cw)],
                acc_out.at[c].at[pl.ds(q * QN + 2432, 68), pl.ds(c0, cw)]))
            plsc.subcore_barrier()


_accum = functools.partial(
    pl.kernel,
    out_type=jax.ShapeDtypeStruct((2, N_NODES, HX), _f32),
    mesh=_mesh,
    compiler_params=pltpu.CompilerParams(needs_layout_passes=False,
                                         use_tc_tiling_on_sc=False),
    scratch_types=[
        pltpu.VMEM_SHARED((N_NODES, 136), _f32),
        pltpu.VMEM_SHARED((SLABR, 136), _f32),
        pltpu.VMEM((CHUNK, 136), _f32),
        pltpu.VMEM((CHUNK, 136), _f32),
        pltpu.VMEM((NCH, CHUNK), _i32),
        pltpu.VMEM((NCH, CHUNK), _i32),
        pltpu.VMEM((16,), _i32),
        pltpu.SemaphoreType.DMA,
        pltpu.SemaphoreType.DMA,
    ],
)(_accum_body)


def _msgs_gru_body(a0_ref, a1_ref, h_ref, W_ref, Bt_ref, K_ref, RK_ref,
                   Bg_ref, hx_ref, h_out_ref):
    a0 = a0_ref[:, :HIDDEN]
    a1 = a1_ref[:, :HIDDEN]
    c0 = a0_ref[:, HIDDEN:HIDDEN + 1]
    c1 = a1_ref[:, HIDDEN:HIDDEN + 1]
    h = h_ref[...]
    msg = (jnp.dot(a0, W_ref[0], preferred_element_type=_f32)
           + jnp.dot(a1, W_ref[1], preferred_element_type=_f32)
           + c0 * Bt_ref[0][None, :] + c1 * Bt_ref[1][None, :])
    mx = jnp.dot(msg, K_ref[...], preferred_element_type=_f32) + Bg_ref[0][None, :]
    mh = jnp.dot(h, RK_ref[...], preferred_element_type=_f32) + Bg_ref[1][None, :]
    z = jax.nn.sigmoid(mx[:, :HIDDEN] + mh[:, :HIDDEN])
    r = jax.nn.sigmoid(mx[:, HIDDEN:2 * HIDDEN] + mh[:, HIDDEN:2 * HIDDEN])
    hh = jnp.tanh(mx[:, 2 * HIDDEN:] + r * mh[:, 2 * HIDDEN:])
    hnew = z * h + (1.0 - z) * hh
    h_out_ref[...] = hnew
    hx_ref[:, :HIDDEN] = hnew
    col = lax.broadcasted_iota(_i32, (hnew.shape[0], HX - HIDDEN), 1)
    hx_ref[:, HIDDEN:] = jnp.where(col == 0, 1.0, 0.0).astype(_f32)


def _msgs_gru(a0, a1, h, W, Bt, K, RK, Bg):
    blk = lambda cols: pl.BlockSpec((BN, cols), lambda i: (i, 0))
    full = lambda shp: pl.BlockSpec(shp, lambda i: tuple(0 for _ in shp))
    return pl.pallas_call(
        _msgs_gru_body,
        grid=(N_NODES // BN,),
        in_specs=[
            blk(HX), blk(HX), blk(HIDDEN),
            full((2, HIDDEN, HIDDEN)), full((2, HIDDEN)),
            full((HIDDEN, 3 * HIDDEN)), full((HIDDEN, 3 * HIDDEN)),
            full((2, 3 * HIDDEN)),
        ],
        out_specs=[blk(HX), blk(HIDDEN)],
        out_shape=[
            jax.ShapeDtypeStruct((N_NODES, HX), _f32),
            jax.ShapeDtypeStruct((N_NODES, HIDDEN), _f32),
        ],
    )(a0, a1, h, W, Bt, K, RK, Bg)


def kernel(states, edge_ids, W, Bt, K, RK, Bg):
    pad = jnp.zeros((16,), _i32)
    etype = jnp.concatenate([edge_ids[:, 0], pad])
    src = jnp.concatenate([edge_ids[:, 1], pad])
    tgt = jnp.concatenate([edge_ids[:, 2], pad])
    srcl, locl, nch = _prep(etype, src, tgt)
    srcl = srcl.reshape(NW, NT, NCH, CHUNK)
    locl = locl.reshape(NW, NT, NCH, CHUNK)
    zrows = jnp.zeros((ZR, 136), _f32)
    hx = jnp.concatenate(
        [states,
         jnp.broadcast_to(jnp.float32(1.0), (N_NODES, 1)),
         jnp.zeros((N_NODES, HX - HIDDEN - 1), _f32)], axis=1)
    h = states
    for _ in range(2):
        acc = _accum(hx, srcl, locl, nch, zrows)
        hx, h = _msgs_gru(acc[0], acc[1], h, W[0], Bt[0], K[0], RK[0], Bg[0])
    return h


# speedup vs baseline: 4.9334x; 1.8243x over previous
import functools

import jax
import jax.numpy as jnp
from jax import lax
from jax.experimental import pallas as pl
from jax.experimental.pallas import tpu as pltpu
from jax.experimental.pallas import tpu_sc as plsc

N_NODES = 10000
HIDDEN = 256
HX = 272
BN = 1000
E = 160000
NW = 32
EW = E // NW
EWP = EW + 16
HALF = N_NODES // 2
CHUNK = 128
NCH = 13
SLABR = 5120
JUNK = 5000
ZR = 128

_mesh = plsc.VectorSubcoreMesh(core_axis_name="c", subcore_axis_name="s")
_i32 = jnp.int32
_f32 = jnp.float32


def _prep_body(et_hbm, sr_hbm, tg_hbm, srcl_out, locl_out, nch_out,
               et_v, sr_v, tg_v, sbuf, lbuf, nbuf):
    w = lax.axis_index("s") * 2 + lax.axis_index("c")
    base = w * EW
    pltpu.sync_copy(et_hbm.at[pl.ds(base, EWP)], et_v)
    pltpu.sync_copy(sr_hbm.at[pl.ds(base, EWP)], sr_v)
    pltpu.sync_copy(tg_hbm.at[pl.ds(base, EWP)], tg_v)

    lane = lax.iota(_i32, 16)
    nsteps = EWP // 16

    def step(i, cnts):
        off = i * 16
        et = et_v[pl.ds(off, 16)]
        sr = sr_v[pl.ds(off, 16)]
        tg = tg_v[pl.ds(off, 16)]
        valid = (off + lane) < EW
        hf = jnp.where(tg >= HALF, 1, 0)
        tsk = et * 2 + hf
        loc = tg - hf * HALF
        new = []
        for t in range(4):
            m = valid & (tsk == t)
            mi = jnp.where(m, 1, 0)
            cs = lax.cumsum(mi)
            pos = cnts[t] + cs - mi
            pk = pos // 16
            pcol = pos % 16
            t16 = jnp.full((16,), t, _i32)
            plsc.store_scatter(sbuf, [t16, pk, pcol], sr, mask=m)
            plsc.store_scatter(lbuf, [t16, pk, pcol], loc, mask=m)
            new.append(cnts[t] + jnp.sum(mi))
        return tuple(new)

    z = jnp.asarray(0, _i32)
    cnts = lax.fori_loop(0, nsteps, step, (z, z, z, z))

    nch_vec = jnp.zeros((16,), _i32)
    zeros16 = jnp.zeros((16,), _i32)
    junk16 = jnp.full((16,), JUNK, _i32)
    for t in range(4):
        cnt = cnts[t]
        nch = (cnt + (CHUNK - 1)) // CHUNK
        end = nch * CHUNK
        t16 = jnp.full((16,), t, _i32)
        for p in range(CHUNK // 16):
            idx = cnt + p * 16 + lane
            m = idx < end
            pk = idx // 16
            pcol = idx % 16
            plsc.store_scatter(sbuf, [t16, pk, pcol], zeros16, mask=m)
            plsc.store_scatter(lbuf, [t16, pk, pcol], junk16, mask=m)
        nch_vec = jnp.where(lane == t, nch, nch_vec)

    nbuf[...] = nch_vec
    pltpu.sync_copy(nbuf, nch_out.at[w])
    pltpu.sync_copy(sbuf, srcl_out.at[w])
    pltpu.sync_copy(lbuf, locl_out.at[w])


_prep = functools.partial(
    pl.kernel,
    out_type=[
        jax.ShapeDtypeStruct((NW, 4, NCH * 8, 16), _i32),
        jax.ShapeDtypeStruct((NW, 4, NCH * 8, 16), _i32),
        jax.ShapeDtypeStruct((NW, 16), _i32),
    ],
    mesh=_mesh,
    compiler_params=pltpu.CompilerParams(needs_layout_passes=False),
    scratch_types=[
        pltpu.VMEM((EWP,), _i32),
        pltpu.VMEM((EWP,), _i32),
        pltpu.VMEM((EWP,), _i32),
        pltpu.VMEM((4, NCH * 8, 16), _i32),
        pltpu.VMEM((4, NCH * 8, 16), _i32),
        pltpu.VMEM((16,), _i32),
    ],
)(_prep_body)


def _accum_body(hx_hbm, srcl_hbm, locl_hbm, nch_hbm, z_hbm, acc_out,
                table, slab, g0, g1, sidx, lidx, cbuf, sem0, sem1):
    c = lax.axis_index("c")
    s = lax.axis_index("s")
    lane = lax.iota(_i32, 16)
    gbufs = (g0, g1)
    sems = (sem0, sem1)

    for (c0, cw) in ((0, 96), (96, 96), (192, 80)):
        tr = s * 624
        pltpu.sync_copy(hx_hbm.at[pl.ds(tr, 624), pl.ds(c0, cw)],
                        table.at[pl.ds(tr, 624), pl.ds(0, cw)])
        pl.when(s == 0)(lambda c0=c0, cw=cw: pltpu.sync_copy(
            hx_hbm.at[pl.ds(9984, 16), pl.ds(c0, cw)],
            table.at[pl.ds(9984, 16), pl.ds(0, cw)]))
        plsc.subcore_barrier()

        for hh in range(2):
            r0 = s * 320
            pltpu.sync_copy(z_hbm.at[pl.ds(0, 128)], slab.at[pl.ds(r0, 128)])
            pltpu.sync_copy(z_hbm.at[pl.ds(0, 128)],
                            slab.at[pl.ds(r0 + 128, 128)])
            pltpu.sync_copy(z_hbm.at[pl.ds(0, 64)],
                            slab.at[pl.ds(r0 + 256, 64)])
            plsc.subcore_barrier()

            j = c * 2 + hh
            for wsel in range(2):
                w = s * 2 + wsel
                pltpu.sync_copy(nch_hbm.at[w], cbuf)
                n = jnp.sum(jnp.where(lane == j, cbuf[...], 0))
                pltpu.sync_copy(srcl_hbm.at[w, j], sidx)
                pltpu.sync_copy(locl_hbm.at[w, j], lidx)

                descs = [
                    pltpu.make_async_copy(table.at[sidx.at[k]],
                                          gbufs[k % 2], sems[k % 2])
                    for k in range(NCH)
                ]
                pl.when(n > 0)(lambda: descs[0].start())
                for k in range(NCH):
                    if k + 1 < NCH:
                        pl.when(k + 1 < n)(lambda k=k: descs[k + 1].start())

                    def chunk(k=k):
                        descs[k].wait()
                        pltpu.sync_copy(gbufs[k % 2], slab.at[lidx.at[k]],
                                        add=True)
                    pl.when(k < n)(chunk)

            plsc.subcore_barrier()
            wr0 = s * 312
            pltpu.sync_copy(
                slab.at[pl.ds(wr0, 312), pl.ds(0, cw)],
                acc_out.at[c].at[pl.ds(hh * HALF + wr0, 312), pl.ds(c0, cw)])
            pl.when(s == 0)(lambda c0=c0, cw=cw, hh=hh: pltpu.sync_copy(
                slab.at[pl.ds(4992, 8), pl.ds(0, cw)],
                acc_out.at[c].at[pl.ds(hh * HALF + 4992, 8), pl.ds(c0, cw)]))
            plsc.subcore_barrier()


_accum = functools.partial(
    pl.kernel,
    out_type=jax.ShapeDtypeStruct((2, N_NODES, HX), _f32),
    mesh=_mesh,
    compiler_params=pltpu.CompilerParams(needs_layout_passes=False,
                                         use_tc_tiling_on_sc=False),
    scratch_types=[
        pltpu.VMEM_SHARED((N_NODES, 96), _f32),
        pltpu.VMEM_SHARED((SLABR, 96), _f32),
        pltpu.VMEM((CHUNK, 96), _f32),
        pltpu.VMEM((CHUNK, 96), _f32),
        pltpu.VMEM((NCH, CHUNK), _i32),
        pltpu.VMEM((NCH, CHUNK), _i32),
        pltpu.VMEM((16,), _i32),
        pltpu.SemaphoreType.DMA,
        pltpu.SemaphoreType.DMA,
    ],
)(_accum_body)


def _msgs_gru_body(a0_ref, a1_ref, h_ref, W_ref, Bt_ref, K_ref, RK_ref,
                   Bg_ref, hx_ref, h_out_ref):
    a0 = a0_ref[:, :HIDDEN]
    a1 = a1_ref[:, :HIDDEN]
    c0 = a0_ref[:, HIDDEN:HIDDEN + 1]
    c1 = a1_ref[:, HIDDEN:HIDDEN + 1]
    h = h_ref[...]
    msg = (jnp.dot(a0, W_ref[0], preferred_element_type=_f32)
           + jnp.dot(a1, W_ref[1], preferred_element_type=_f32)
           + c0 * Bt_ref[0][None, :] + c1 * Bt_ref[1][None, :])
    mx = jnp.dot(msg, K_ref[...], preferred_element_type=_f32) + Bg_ref[0][None, :]
    mh = jnp.dot(h, RK_ref[...], preferred_element_type=_f32) + Bg_ref[1][None, :]
    z = jax.nn.sigmoid(mx[:, :HIDDEN] + mh[:, :HIDDEN])
    r = jax.nn.sigmoid(mx[:, HIDDEN:2 * HIDDEN] + mh[:, HIDDEN:2 * HIDDEN])
    hh = jnp.tanh(mx[:, 2 * HIDDEN:] + r * mh[:, 2 * HIDDEN:])
    hnew = z * h + (1.0 - z) * hh
    h_out_ref[...] = hnew
    hx_ref[:, :HIDDEN] = hnew
    col = lax.broadcasted_iota(_i32, (hnew.shape[0], HX - HIDDEN), 1)
    hx_ref[:, HIDDEN:] = jnp.where(col == 0, 1.0, 0.0).astype(_f32)


def _msgs_gru(a0, a1, h, W, Bt, K, RK, Bg):
    blk = lambda cols: pl.BlockSpec((BN, cols), lambda i: (i, 0))
    full = lambda shp: pl.BlockSpec(shp, lambda i: tuple(0 for _ in shp))
    return pl.pallas_call(
        _msgs_gru_body,
        grid=(N_NODES // BN,),
        in_specs=[
            blk(HX), blk(HX), blk(HIDDEN),
            full((2, HIDDEN, HIDDEN)), full((2, HIDDEN)),
            full((HIDDEN, 3 * HIDDEN)), full((HIDDEN, 3 * HIDDEN)),
            full((2, 3 * HIDDEN)),
        ],
        out_specs=[blk(HX), blk(HIDDEN)],
        out_shape=[
            jax.ShapeDtypeStruct((N_NODES, HX), _f32),
            jax.ShapeDtypeStruct((N_NODES, HIDDEN), _f32),
        ],
    )(a0, a1, h, W, Bt, K, RK, Bg)


def kernel(states, edge_ids, W, Bt, K, RK, Bg):
    pad = jnp.zeros((16,), _i32)
    etype = jnp.concatenate([edge_ids[:, 0], pad])
    src = jnp.concatenate([edge_ids[:, 1], pad])
    tgt = jnp.concatenate([edge_ids[:, 2], pad])
    srcl, locl, nch = _prep(etype, src, tgt)
    srcl = srcl.reshape(NW, 4, NCH, CHUNK)
    locl = locl.reshape(NW, 4, NCH, CHUNK)
    zrows = jnp.zeros((ZR, 96), _f32)
    hx = jnp.concatenate(
        [states,
         jnp.broadcast_to(jnp.float32(1.0), (N_NODES, 1)),
         jnp.zeros((N_NODES, HX - HIDDEN - 1), _f32)], axis=1)
    h = states
    for _ in range(2):
        acc = _accum(hx, srcl, locl, nch, zrows)
        hx, h = _msgs_gru(acc[0], acc[1], h, W[0], Bt[0], K[0], RK[0], Bg[0])
    return h
